# Initial kernel scaffold; baseline (speedup 1.0000x reference)
import functools

import jax
import jax.numpy as jnp
from jax import lax
from jax.experimental import pallas as pl
from jax.experimental.pallas import tpu as pltpu
from jax.experimental.pallas import tpu_sc as plsc

NC, NS, L = 2, 16, 16
NW = NC * NS
B, S, D = 4, 2048, 1024
S_PER_W = S // NW
CHUNK = 16
NSC = S_PER_W // CHUNK
NCH = NSC * B
NBUF = 4
LPR = D // L


def _body(x_hbm, tok_hbm, pos_hbm, out_hbm, idx_v, pos_bufs, tok_bufs,
          in_sems, out_sems, pos_sems):
    wid = lax.axis_index("s") * NC + lax.axis_index("c")
    s0 = pl.multiple_of(wid * S_PER_W, S_PER_W)

    for b in range(B):
        pltpu.sync_copy(x_hbm.at[b, pl.ds(s0, S_PER_W)], idx_v.at[b])

    out_handles = [None] * NBUF

    def issue_gather(c):
        sc, b = divmod(c, B)
        i = c % NBUF
        if out_handles[i] is not None:
            out_handles[i].wait()
            out_handles[i] = None
        idx_slice = idx_v.at[b, pl.ds(sc * CHUNK, CHUNK)]
        return pltpu.async_copy(tok_hbm.at[idx_slice], tok_bufs.at[i],
                                in_sems.at[i])

    def issue_pos(sc):
        src = pos_hbm.at[pl.ds(s0 + sc * CHUNK, CHUNK)]
        return pltpu.async_copy(src, pos_bufs.at[sc % 2], pos_sems.at[sc % 2])

    pos_handles = {0: issue_pos(0)}
    gather_handles = {}
    for c in range(min(2, NCH)):
        gather_handles[c] = issue_gather(c)

    for c in range(NCH):
        sc, b = divmod(c, B)
        i = c % NBUF
        if c + 2 < NCH:
            gather_handles[c + 2] = issue_gather(c + 2)
        gather_handles.pop(c).wait()
        if b == 0:
            if sc + 1 < NSC:
                pos_handles[sc + 1] = issue_pos(sc + 1)
            h = pos_handles.pop(sc, None)
            if h is not None:
                h.wait()
        tok = tok_bufs.at[i]
        pos = pos_bufs.at[sc % 2]

        @plsc.parallel_loop(0, CHUNK * LPR, unroll=4)
        def _(j):
            r = j // LPR
            col = (j - r * LPR) * L
            tok[r, pl.ds(col, L)] = tok[r, pl.ds(col, L)] + pos[r, pl.ds(col, L)]

        dst = out_hbm.at[b, pl.ds(s0 + sc * CHUNK, CHUNK)]
        out_handles[i] = pltpu.async_copy(tok, dst, out_sems.at[i])

    for h in out_handles:
        if h is not None:
            h.wait()


@jax.jit
def kernel(x, token_table, position_table):
    mesh = plsc.VectorSubcoreMesh(core_axis_name="c", subcore_axis_name="s")
    run = pl.kernel(
        _body,
        out_type=jax.ShapeDtypeStruct((B, S, D), jnp.float32),
        mesh=mesh,
        scratch_types=dict(
            idx_v=pltpu.VMEM((B, S_PER_W), jnp.int32),
            pos_bufs=pltpu.VMEM((2, CHUNK, D), jnp.float32),
            tok_bufs=pltpu.VMEM((NBUF, CHUNK, D), jnp.float32),
            in_sems=pltpu.SemaphoreType.DMA((NBUF,)),
            out_sems=pltpu.SemaphoreType.DMA((NBUF,)),
            pos_sems=pltpu.SemaphoreType.DMA((2,)),
        ),
    )
    return run(x.astype(jnp.int32), token_table, position_table)


# speedup vs baseline: 1.6155x; 1.6155x over previous
import functools

import jax
import jax.numpy as jnp
from jax import lax
from jax.experimental import pallas as pl
from jax.experimental.pallas import tpu as pltpu
from jax.experimental.pallas import tpu_sc as plsc

NC, NS, L = 2, 16, 16
NW = NC * NS
B, S, D = 4, 2048, 1024
S_PER_W = S // NW
CHUNK = 16
NSC = S_PER_W // CHUNK
NCH = NSC * B
NBUF = 4
LPR = D // L


def _body(x_hbm, tok_hbm, pos_hbm, out_hbm, idx_v, pos_bufs, tok_bufs,
          in_sems, out_sems, pos_sems):
    wid = lax.axis_index("s") * NC + lax.axis_index("c")
    s0 = pl.multiple_of(wid * S_PER_W, S_PER_W)

    for b in range(B):
        pltpu.sync_copy(x_hbm.at[b, pl.ds(s0, S_PER_W)], idx_v.at[b])

    out_handles = [None] * NBUF

    def issue_gather(c):
        sc, b = divmod(c, B)
        i = c % NBUF
        if out_handles[i] is not None:
            out_handles[i].wait()
            out_handles[i] = None
        idx_vec = idx_v[b, pl.ds(sc * CHUNK, CHUNK)]
        return pltpu.async_copy(tok_hbm.at[idx_vec], tok_bufs.at[i],
                                in_sems.at[i])

    def issue_pos(sc):
        src = pos_hbm.at[pl.ds(s0 + sc * CHUNK, CHUNK)]
        return pltpu.async_copy(src, pos_bufs.at[sc % 2], pos_sems.at[sc % 2])

    pos_handles = {0: issue_pos(0)}
    gather_handles = {}
    for c in range(min(2, NCH)):
        gather_handles[c] = issue_gather(c)

    for c in range(NCH):
        sc, b = divmod(c, B)
        i = c % NBUF
        if c + 2 < NCH:
            gather_handles[c + 2] = issue_gather(c + 2)
        gather_handles.pop(c).wait()
        if b == 0:
            if sc + 1 < NSC:
                pos_handles[sc + 1] = issue_pos(sc + 1)
            h = pos_handles.pop(sc, None)
            if h is not None:
                h.wait()
        tok = tok_bufs.at[i]
        pos = pos_bufs.at[sc % 2]

        @plsc.parallel_loop(0, CHUNK * LPR, unroll=4)
        def _(j):
            r = j // LPR
            col = (j - r * LPR) * L
            tok[r, pl.ds(col, L)] = tok[r, pl.ds(col, L)] + pos[r, pl.ds(col, L)]

        dst = out_hbm.at[b, pl.ds(s0 + sc * CHUNK, CHUNK)]
        out_handles[i] = pltpu.async_copy(tok, dst, out_sems.at[i])

    for h in out_handles:
        if h is not None:
            h.wait()


@jax.jit
def kernel(x, token_table, position_table):
    mesh = plsc.VectorSubcoreMesh(core_axis_name="c", subcore_axis_name="s",
                                  num_cores=NC, num_subcores=NS)
    run = pl.kernel(
        _body,
        out_type=jax.ShapeDtypeStruct((B, S, D), jnp.float32),
        mesh=mesh,
        scratch_types=dict(
            idx_v=pltpu.VMEM((B, S_PER_W), jnp.int32),
            pos_bufs=pltpu.VMEM((2, CHUNK, D), jnp.float32),
            tok_bufs=pltpu.VMEM((NBUF, CHUNK, D), jnp.float32),
            in_sems=pltpu.SemaphoreType.DMA((NBUF,)),
            out_sems=pltpu.SemaphoreType.DMA((NBUF,)),
            pos_sems=pltpu.SemaphoreType.DMA((2,)),
        ),
    )
    return run(x.astype(jnp.int32), token_table, position_table)
